# Initial kernel scaffold; baseline (speedup 1.0000x reference)
import jax
import jax.numpy as jnp
from jax.experimental import pallas as pl


def kernel(x0, x1, edge_index0, edge_index1, W1_0, a1s_0, a1d_0, b1_0, W2_0, a2s_0, a2d_0, b2_0, W3_0, a3s_0, a3d_0, b3_0, W1_1, a1s_1, a1d_1, b1_1, W2_1, a2s_1, a2d_1, b2_1, W3_1, a3s_1, a3d_1, b3_1):
    raise NotImplementedError("write your pallas kernel here")


# speedup vs baseline: 1.0456x; 1.0456x over previous
import functools

import jax
import jax.numpy as jnp
from jax.experimental import pallas as pl
from jax.experimental.pallas import tpu as pltpu

N = 10000
E = 320000
HEADS = 3


def _mm_body(x_ref, w_ref, o_ref):
    o_ref[...] = jnp.dot(x_ref[...], w_ref[...],
                         preferred_element_type=jnp.float32)


@functools.partial(jax.jit, static_argnames=("bm",))
def _matmul(x, w, bm=400):
    n, k = x.shape
    m = w.shape[1]
    grid = (n // bm,)
    return pl.pallas_call(
        _mm_body,
        grid=grid,
        in_specs=[
            pl.BlockSpec((bm, k), lambda i: (i, 0)),
            pl.BlockSpec((k, m), lambda i: (0, 0)),
        ],
        out_specs=pl.BlockSpec((bm, m), lambda i: (i, 0)),
        out_shape=jax.ShapeDtypeStruct((n, m), jnp.float32),
    )(x, w)


def _gat_conv(x, src, dst, W, a_src, a_dst, b, concat):
    n = x.shape[0]
    heads, c = a_src.shape
    h = _matmul(x, W).reshape(n, heads, c)
    alpha_src = (h * a_src[None, :, :]).sum(-1)
    alpha_dst = (h * a_dst[None, :, :]).sum(-1)
    e = jax.nn.leaky_relu(alpha_src[src] + alpha_dst[dst], negative_slope=0.2)
    ex = jnp.exp(e)
    den = jax.ops.segment_sum(ex, dst, num_segments=n)
    alpha = ex / (den[dst] + 1e-16)
    out = jax.ops.segment_sum(h[src] * alpha[:, :, None], dst, num_segments=n)
    if concat:
        out = out.reshape(n, heads * c)
    else:
        out = out.mean(axis=1)
    return out + b


def kernel(x0, x1, edge_index0, edge_index1, W1_0, a1s_0, a1d_0, b1_0, W2_0, a2s_0, a2d_0, b2_0, W3_0, a3s_0, a3d_0, b3_0, W1_1, a1s_1, a1d_1, b1_1, W2_1, a2s_1, a2d_1, b2_1, W3_1, a3s_1, a3d_1, b3_1):
    params = (
        (W1_0, a1s_0, a1d_0, b1_0, W2_0, a2s_0, a2d_0, b2_0, W3_0, a3s_0, a3d_0, b3_0),
        (W1_1, a1s_1, a1d_1, b1_1, W2_1, a2s_1, a2d_1, b2_1, W3_1, a3s_1, a3d_1, b3_1),
    )
    loop = jnp.arange(N, dtype=edge_index0.dtype)
    outs = []
    for x, ei, p in ((x0, edge_index0, params[0]), (x1, edge_index1, params[1])):
        (W1, a1s, a1d, b1, W2, a2s, a2d, b2, W3, a3s, a3d, b3) = p
        src = jnp.concatenate([ei[0], loop])
        dst = jnp.concatenate([ei[1], loop])
        h = jax.nn.elu(_gat_conv(x, src, dst, W1, a1s, a1d, b1, True))
        h = jax.nn.elu(_gat_conv(h, src, dst, W2, a2s, a2d, b2, True))
        h = jax.nn.elu(_gat_conv(h, src, dst, W3, a3s, a3d, b3, False))
        outs.append(h)
    return jnp.stack(outs)


# speedup vs baseline: 9.2484x; 8.8453x over previous
import functools

import jax
import jax.numpy as jnp
from jax import lax
from jax.experimental import pallas as pl
from jax.experimental.pallas import tpu as pltpu
from jax.experimental.pallas import tpu_sc as plsc

N = 10000
E = 320000
HEADS = 3
NH = N * HEADS
EL = E + N
BLK = 128
NBLK = 162
NSC = 16
EPT = BLK * NBLK
E2 = EPT * NSC
F = 128
PASSES = 6
OPP = PASSES + 1
NROW = 640
ACCN = NROW * NSC
D = PASSES * F
BLK2 = 2592
NB2 = EPT // BLK2


def _mm_body(x_ref, w_ref, o_ref):
    o_ref[...] = jnp.dot(x_ref[...], w_ref[...],
                         preferred_element_type=jnp.float32)


@functools.partial(jax.jit, static_argnames=("bm",))
def _matmul(x, w, bm=400):
    n, k = x.shape
    m = w.shape[1]
    return pl.pallas_call(
        _mm_body,
        grid=(n // bm,),
        in_specs=[
            pl.BlockSpec((bm, k), lambda i: (i, 0)),
            pl.BlockSpec((k, m), lambda i: (0, 0)),
        ],
        out_specs=pl.BlockSpec((bm, m), lambda i: (i, 0)),
        out_shape=jax.ShapeDtypeStruct((n, m), jnp.float32),
    )(x, w)


def _ex_body(atabs, atabd, srcp, dstp, exq,
             asrc_t, adst_t, src_v, dst_v, exb_v):
    c = lax.axis_index("c")
    s = lax.axis_index("s")
    pltpu.sync_copy(atabs.at[pl.ds(c * NH, NH)], asrc_t)
    pltpu.sync_copy(atabd.at[pl.ds(c * NH, NH)], adst_t)
    iot = lax.iota(jnp.int32, 16)

    def blk(b, _):
        eb = s * EPT + b * BLK2
        pltpu.sync_copy(srcp.at[pl.ds(c * E2 + eb, BLK2)], src_v)
        pltpu.sync_copy(dstp.at[pl.ds(c * E2 + eb, BLK2)], dst_v)

        def chunk(k, _):
            s3 = src_v[pl.ds(k * 16, 16)] * 3
            d3 = dst_v[pl.ds(k * 16, 16)] * 3
            valid = (eb + k * 16 + iot) < EL
            lidx = (k * 16 + iot) * 3
            for h in range(HEADS):
                a1 = plsc.load_gather(asrc_t, [s3 + h])
                a2 = plsc.load_gather(adst_t, [d3 + h])
                sm = a1 + a2
                e = jnp.where(sm > 0, sm, sm * jnp.float32(0.2))
                ex = jnp.where(valid, jnp.exp(e), jnp.float32(0.0))
                plsc.store_scatter(exb_v, [lidx + h], ex)
            return 0
        lax.fori_loop(0, BLK2 // 16, chunk, 0)
        pltpu.sync_copy(
            exb_v, exq.at[pl.ds((c * E2 + eb) * HEADS, BLK2 * HEADS)])
        return 0
    lax.fori_loop(0, NB2, blk, 0)


@jax.jit
def _exmap(atabs, atabd, srcp, dstp):
    return pl.kernel(
        _ex_body,
        out_type=jax.ShapeDtypeStruct((2 * E2 * HEADS,), jnp.float32),
        mesh=plsc.VectorSubcoreMesh(core_axis_name="c", subcore_axis_name="s"),
        compiler_params=pltpu.CompilerParams(needs_layout_passes=False),
        scratch_types=[
            pltpu.VMEM((NH,), jnp.float32),
            pltpu.VMEM((NH,), jnp.float32),
            pltpu.VMEM((BLK2,), jnp.int32),
            pltpu.VMEM((BLK2,), jnp.int32),
            pltpu.VMEM((BLK2 * HEADS,), jnp.float32),
        ],
    )(atabs, atabd, srcp, dstp)


def _agg_body(hc2, srcp, dstp, alphap, occ,
              idx_v, idxo_v, dst_v, alpha_v, rows_v, stage_v, zero_v,
              acc_sh, sem):
    c = lax.axis_index("c")
    s = lax.axis_index("s")

    def zrow(r, _):
        for g in range(F // 16):
            zero_v[r, pl.ds(g * 16, 16)] = jnp.zeros((16,), jnp.float32)
        return 0
    lax.fori_loop(0, NROW // 8, zrow, 0)

    for p in range(OPP):
        for z in range(8):
            pltpu.sync_copy(zero_v,
                            acc_sh.at[pl.ds(s * NROW + z * (NROW // 8),
                                            NROW // 8)])
        plsc.subcore_barrier()

        if p < PASSES:
            lp = c * PASSES + p
            h = (p * F) // 256

            def blk_body(b, _):
                eb = s * EPT + b * BLK
                pltpu.sync_copy(srcp.at[pl.ds(c * E2 + eb, BLK)], idx_v)
                pltpu.sync_copy(dstp.at[pl.ds(c * E2 + eb, BLK)], dst_v)
                pltpu.sync_copy(
                    alphap.at[pl.ds((c * E2 + eb) * HEADS, BLK * HEADS)],
                    alpha_v.at[pl.ds(0, BLK * HEADS)])
                for k in range(BLK // 16):
                    idxo_v[pl.ds(k * 16, 16)] = (
                        idx_v[pl.ds(k * 16, 16)] + lp * N)
                pltpu.async_copy(hc2.at[idxo_v], rows_v, sem).wait()

                def j_body(j, _):
                    ab = plsc.load_gather(
                        alpha_v, [jnp.full((16,), j * HEADS + h, jnp.int32)])
                    for g in range(F // 16):
                        stage_v[j, pl.ds(g * 16, 16)] = (
                            rows_v[j, pl.ds(g * 16, 16)] * ab)
                    return 0
                lax.fori_loop(0, BLK, j_body, 0)
                pltpu.sync_copy(stage_v, acc_sh.at[dst_v], add=True)
                return 0
            lax.fori_loop(0, NBLK, blk_body, 0)
        else:
            def zst(j, _):
                for g in range(HEADS, F // 16):
                    stage_v[j, pl.ds(g * 16, 16)] = jnp.zeros((16,),
                                                              jnp.float32)
                return 0
            lax.fori_loop(0, BLK, zst, 0)

            def blk_body_d(b, _):
                eb = s * EPT + b * BLK
                pltpu.sync_copy(dstp.at[pl.ds(c * E2 + eb, BLK)], dst_v)
                pltpu.sync_copy(
                    alphap.at[pl.ds((c * E2 + eb) * HEADS, BLK * HEADS)],
                    alpha_v.at[pl.ds(0, BLK * HEADS)])

                def j_body(j, _):
                    for h in range(HEADS):
                        ab = plsc.load_gather(
                            alpha_v,
                            [jnp.full((16,), j * HEADS + h, jnp.int32)])
                        stage_v[j, pl.ds(h * 16, 16)] = ab
                    return 0
                lax.fori_loop(0, BLK, j_body, 0)
                pltpu.sync_copy(stage_v, acc_sh.at[dst_v], add=True)
                return 0
            lax.fori_loop(0, NBLK, blk_body_d, 0)

        plsc.subcore_barrier()
        pltpu.sync_copy(
            acc_sh.at[pl.ds(s * NROW, NROW)],
            occ.at[pl.ds((c * OPP + p) * ACCN + s * NROW, NROW)])
        plsc.subcore_barrier()


@jax.jit
def _aggregate(hc2, srcp, dstp, alphap):
    return pl.kernel(
        _agg_body,
        out_type=jax.ShapeDtypeStruct((2 * OPP * ACCN, F), jnp.float32),
        mesh=plsc.VectorSubcoreMesh(core_axis_name="c", subcore_axis_name="s"),
        compiler_params=pltpu.CompilerParams(needs_layout_passes=False),
        scratch_types=[
            pltpu.VMEM((BLK,), jnp.int32),
            pltpu.VMEM((BLK,), jnp.int32),
            pltpu.VMEM((BLK,), jnp.int32),
            pltpu.VMEM((BLK * HEADS + 16,), jnp.float32),
            pltpu.VMEM((BLK, F), jnp.float32),
            pltpu.VMEM((BLK, F), jnp.float32),
            pltpu.VMEM((NROW // 8, F), jnp.float32),
            pltpu.VMEM_SHARED((ACCN, F), jnp.float32),
            pltpu.SemaphoreType.DMA,
        ],
    )(hc2, srcp, dstp, alphap)


def kernel(x0, x1, edge_index0, edge_index1, W1_0, a1s_0, a1d_0, b1_0, W2_0, a2s_0, a2d_0, b2_0, W3_0, a3s_0, a3d_0, b3_0, W1_1, a1s_1, a1d_1, b1_1, W2_1, a2s_1, a2d_1, b2_1, W3_1, a3s_1, a3d_1, b3_1):
    params = (
        (W1_0, a1s_0, a1d_0, b1_0, W2_0, a2s_0, a2d_0, b2_0, W3_0, a3s_0, a3d_0, b3_0),
        (W1_1, a1s_1, a1d_1, b1_1, W2_1, a2s_1, a2d_1, b2_1, W3_1, a3s_1, a3d_1, b3_1),
    )
    loop = jnp.arange(N, dtype=edge_index0.dtype)
    srcp, dstp = [], []
    for ei in (edge_index0, edge_index1):
        src = jnp.concatenate([ei[0], loop]).astype(jnp.int32)
        dst = jnp.concatenate([ei[1], loop]).astype(jnp.int32)
        srcp.append(jnp.pad(src, (0, E2 - EL)))
        dstp.append(jnp.pad(dst, (0, E2 - EL)))
    srcp = jnp.concatenate(srcp)
    dstp = jnp.concatenate(dstp)

    hs = [x0, x1]
    for conv in range(3):
        hcs, atabs, atabd = [], [], []
        for i in range(2):
            (W1, a1s, a1d, b1, W2, a2s, a2d, b2, W3, a3s, a3d, b3) = params[i]
            W, a_s, a_d = ((W1, a1s, a1d), (W2, a2s, a2d), (W3, a3s, a3d))[conv]
            h = _matmul(hs[i], W).reshape(N, HEADS, W.shape[1] // HEADS)
            atabs.append((h * a_s[None]).sum(-1).reshape(NH))
            atabd.append((h * a_d[None]).sum(-1).reshape(NH))
            hcs.append(h.reshape(N, PASSES, F).transpose(1, 0, 2))
        hc2 = jnp.stack(hcs).reshape(2 * PASSES * N, F)
        exq = _exmap(jnp.concatenate(atabs), jnp.concatenate(atabd), srcp, dstp)
        occ = _aggregate(hc2, srcp, dstp, exq)
        occ = occ.reshape(2, OPP, ACCN, F)
        feats = (occ[:, :PASSES, :N]
                 .transpose(0, 2, 1, 3).reshape(2, N, D))
        den = occ[:, PASSES, :N, : HEADS * 16 : 16]
        out = feats / (jnp.repeat(den, 256, axis=-1) + 1e-16)
        nh = []
        for i in range(2):
            b = params[i][3 + 4 * conv]
            o = out[i]
            if conv == 2:
                o = o.reshape(N, HEADS, 256).mean(axis=1)
            nh.append(jax.nn.elu(o + b))
        hs = nh
    return jnp.stack(hs)
